# baseline (speedup 1.0000x reference)
import jax
import jax.numpy as jnp
from jax import lax
from jax.experimental import pallas as pl
from jax.experimental.pallas import tpu as pltpu
from jax.experimental.pallas import tpu_sc as plsc

N_TILES = 32
ECHUNK = 128
HID = 128
H3 = 3 * HID


def _fill_const(ref, n_rows, value):
    val = jnp.full((16,), value, jnp.float32)

    def body(i, _):
        for k in range(HID // 16):
            ref[i, pl.ds(k * 16, 16)] = val
        return 0
    lax.fori_loop(0, n_rows, body, 0)


def _zero_acc_slice(zsrc_v, acc_s, s, rows_per_tile):
    for q in range(rows_per_tile // ECHUNK):
        pltpu.sync_copy(
            zsrc_v, acc_s.at[pl.ds(s * rows_per_tile + q * ECHUNK, ECHUNK)])


def _writeback(acc_s, out_hbm, c, s, rows_per_tile):
    npad = acc_s.shape[0]
    pltpu.sync_copy(acc_s.at[pl.ds(s * rows_per_tile, rows_per_tile)],
                    out_hbm.at[pl.ds(c * npad + s * rows_per_tile, rows_per_tile)])


def _deg_body(dst_hbm, out_hbm, ones_v, idx_v, acc_s):
    c = lax.axis_index("c")
    s = lax.axis_index("s")
    w = c * 16 + s
    chunks = idx_v.shape[0]
    rows_per_tile = acc_s.shape[0] // 16

    pltpu.sync_copy(dst_hbm.at[w], idx_v)

    _fill_const(ones_v, ECHUNK, 0.0)
    _zero_acc_slice(ones_v, acc_s, s, rows_per_tile)
    _fill_const(ones_v, ECHUNK, 1.0)
    plsc.subcore_barrier()

    def chunk(j, _):
        pltpu.sync_copy(ones_v, acc_s.at[idx_v.at[j]], add=True)
        return 0
    lax.fori_loop(0, chunks, chunk, 0)
    plsc.subcore_barrier()

    _writeback(acc_s, out_hbm, c, s, rows_per_tile)


def _edge_body(ys_hbm, pk_hbm, out_hbm,
               pk_v, us0, ud0, us1, ud1, rows0, rows1, acc_s, sem0, sem1):
    c = lax.axis_index("c")
    s = lax.axis_index("s")
    w = c * 16 + s
    chunks = pk_v.shape[0]
    n_pairs = chunks // 2
    rows_per_tile = acc_s.shape[0] // 16

    pltpu.sync_copy(pk_hbm.at[w], pk_v)

    _fill_const(rows0, ECHUNK, 0.0)
    _zero_acc_slice(rows0, acc_s, s, rows_per_tile)
    plsc.subcore_barrier()

    def unpack(j, us, ud):
        for k in range(ECHUNK // 16):
            v = pk_v[j, pl.ds(k * 16, 16)]
            us[pl.ds(k * 16, 16)] = jnp.bitwise_and(v, 0xFFFF)
            ud[pl.ds(k * 16, 16)] = lax.shift_right_logical(v, 16)

    unpack(0, us0, ud0)
    pltpu.async_copy(ys_hbm.at[us0], rows0, sem0)
    unpack(1, us1, ud1)

    def pair(k, _):
        j = 2 * k
        pltpu.async_copy(ys_hbm.at[us1], rows1, sem1)
        pltpu.make_async_copy(ys_hbm.at[us0], rows0, sem0).wait()
        pltpu.sync_copy(rows0, acc_s.at[ud0], add=True)

        @pl.when(k < n_pairs - 1)
        def _():
            unpack(j + 2, us0, ud0)
            pltpu.async_copy(ys_hbm.at[us0], rows0, sem0)

        pltpu.make_async_copy(ys_hbm.at[us1], rows1, sem1).wait()
        pltpu.sync_copy(rows1, acc_s.at[ud1], add=True)

        @pl.when(k < n_pairs - 1)
        def _():
            unpack(j + 3, us1, ud1)
        return 0

    lax.fori_loop(0, n_pairs, pair, 0)
    plsc.subcore_barrier()

    _writeback(acc_s, out_hbm, c, s, rows_per_tile)


def _make_sc_calls(npad, chunks):
    mesh = plsc.VectorSubcoreMesh(core_axis_name="c", subcore_axis_name="s")
    deg_call = pl.kernel(
        _deg_body,
        out_type=jax.ShapeDtypeStruct((2 * npad, HID), jnp.float32),
        mesh=mesh,
        scratch_types=[
            pltpu.VMEM((ECHUNK, HID), jnp.float32),
            pltpu.VMEM((chunks, ECHUNK), jnp.int32),
            pltpu.VMEM_SHARED((npad, HID), jnp.float32),
        ],
    )
    edge_call = pl.kernel(
        _edge_body,
        out_type=jax.ShapeDtypeStruct((2 * npad, HID), jnp.float32),
        mesh=mesh,
        scratch_types=[
            pltpu.VMEM((chunks, ECHUNK), jnp.int32),
            pltpu.VMEM((ECHUNK,), jnp.int32),
            pltpu.VMEM((ECHUNK,), jnp.int32),
            pltpu.VMEM((ECHUNK,), jnp.int32),
            pltpu.VMEM((ECHUNK,), jnp.int32),
            pltpu.VMEM((ECHUNK, HID), jnp.float32),
            pltpu.VMEM((ECHUNK, HID), jnp.float32),
            pltpu.VMEM_SHARED((npad, HID), jnp.float32),
            pltpu.SemaphoreType.DMA,
            pltpu.SemaphoreType.DMA,
        ],
    )
    return deg_call, edge_call


def _dinv(degp_ref):
    return lax.rsqrt(degp_ref[0, :, 0] + degp_ref[1, :, 0] + 1.0)


def _prep_body(x_ref, w_ref, degp_ref, o_ref):
    dinv = _dinv(degp_ref)
    y = jnp.dot(x_ref[...], w_ref[...], preferred_element_type=jnp.float32)
    o_ref[...] = y * dinv[:, None]


def _mid_body(acc_ref, ys_ref, degp_ref, b1_ref, w2_ref, o_ref):
    dinv = _dinv(degp_ref)
    h = (acc_ref[0] + acc_ref[1] + ys_ref[...]) * dinv[:, None] + b1_ref[0][None, :]
    h = jnp.maximum(h, 0.0)
    o_ref[...] = jnp.dot(h, w2_ref[...], preferred_element_type=jnp.float32) * dinv[:, None]


def _gi_body(acc_ref, ys_ref, degp_ref, b2_ref, wih_ref, bih_ref, o_ref):
    dinv = _dinv(degp_ref)
    h = (acc_ref[0] + acc_ref[1] + ys_ref[...]) * dinv[:, None] + b2_ref[0][None, :]
    h = jnp.maximum(h, 0.0)
    o_ref[...] = (jnp.dot(h, wih_ref[...], preferred_element_type=jnp.float32)
                  + bih_ref[0][None, :])


TSTEP = 8


def _gru_body(gi_ref, whh_ref, bhh_ref, o_ref, h_ref):
    t = pl.program_id(0)

    @pl.when(t == 0)
    def _():
        h_ref[...] = jnp.zeros_like(h_ref)

    h = h_ref[...]
    whh = whh_ref[...]
    bhh = bhh_ref[0][None, :]
    for i in range(TSTEP):
        g = gi_ref[:, i, :]
        gh = jnp.dot(h, whh, preferred_element_type=jnp.float32) + bhh
        r = jax.nn.sigmoid(g[:, :HID] + gh[:, :HID])
        z = jax.nn.sigmoid(g[:, HID:2 * HID] + gh[:, HID:2 * HID])
        n = jnp.tanh(g[:, 2 * HID:] + r * gh[:, 2 * HID:])
        h = (1.0 - z) * n + z * h
        o_ref[:, i, :] = h
    h_ref[...] = h


def kernel(x, edge_index, W1, b1, W2, b2, W_ih, W_hh, b_ih, b_hh):
    B, T, C = x.shape
    tpad = 512
    npad = B * tpad
    E = edge_index.shape[1]
    chunks = -(-E // (N_TILES * ECHUNK))
    chunks += chunks % 2
    epad = N_TILES * chunks * ECHUNK

    xf = jnp.pad(x, ((0, 0), (0, tpad - T), (0, 0))).reshape(npad, C)
    src = edge_index[0].astype(jnp.int32)
    dst = edge_index[1].astype(jnp.int32)
    srcp = (src // T) * tpad + (src % T)
    dstp = (dst // T) * tpad + (dst % T)
    srcp = jnp.concatenate([srcp, jnp.zeros((epad - E,), jnp.int32)])
    dstp = jnp.concatenate([dstp, jnp.full((epad - E,), T, jnp.int32)])
    dst3 = dstp.reshape(N_TILES, chunks, ECHUNK)
    pk3 = (srcp | (dstp << 16)).reshape(N_TILES, chunks, ECHUNK)

    deg_call, edge_call = _make_sc_calls(npad, chunks)

    rows_blk = 1024
    grid = npad // rows_blk

    def tc_call(body, out_dim, *ops):
        specs = []
        for op in ops:
            if op.shape[0] == 2 and op.ndim == 3:
                specs.append(pl.BlockSpec((2, rows_blk, op.shape[2]),
                                          lambda i: (0, i, 0)))
            elif op.shape[0] == npad:
                specs.append(pl.BlockSpec((rows_blk, op.shape[1]),
                                          lambda i: (i, 0)))
            else:
                specs.append(pl.BlockSpec(op.shape, lambda i, nd=op.ndim: (0,) * nd))
        return pl.pallas_call(
            body,
            grid=(grid,),
            in_specs=specs,
            out_specs=pl.BlockSpec((rows_blk, out_dim), lambda i: (i, 0)),
            out_shape=jax.ShapeDtypeStruct((npad, out_dim), jnp.float32),
        )(*ops)

    degp = deg_call(dst3).reshape(2, npad, HID)
    ys1 = tc_call(_prep_body, HID, xf, W1, degp)
    acc1 = edge_call(ys1, pk3).reshape(2, npad, HID)
    ys2 = tc_call(_mid_body, HID, acc1, ys1, degp, b1.reshape(1, HID), W2)
    acc2 = edge_call(ys2, pk3).reshape(2, npad, HID)
    gi = tc_call(_gi_body, H3, acc2, ys2, degp, b2.reshape(1, HID),
                 W_ih.T, b_ih.reshape(1, H3))

    gi3 = gi.reshape(B, tpad, H3)

    return pl.pallas_call(
        _gru_body,
        grid=(-(-T // TSTEP),),
        in_specs=[
            pl.BlockSpec((B, TSTEP, H3), lambda t: (0, t, 0)),
            pl.BlockSpec((HID, H3), lambda t: (0, 0)),
            pl.BlockSpec((1, H3), lambda t: (0, 0)),
        ],
        out_specs=pl.BlockSpec((B, TSTEP, HID), lambda t: (0, t, 0)),
        out_shape=jax.ShapeDtypeStruct((B, T, HID), jnp.float32),
        scratch_shapes=[pltpu.VMEM((B, HID), jnp.float32)],
    )(gi3, W_hh.T, b_hh.reshape(1, H3))


# speedup vs baseline: 1.3326x; 1.3326x over previous
import jax
import jax.numpy as jnp
from jax import lax
from jax.experimental import pallas as pl
from jax.experimental.pallas import tpu as pltpu
from jax.experimental.pallas import tpu_sc as plsc

N_TILES = 32
ECHUNK = 128
HID = 128
H3 = 3 * HID


def _fill_const(ref, n_rows, value):
    val = jnp.full((16,), value, jnp.float32)

    def body(i, _):
        for k in range(HID // 16):
            ref[i, pl.ds(k * 16, 16)] = val
        return 0
    lax.fori_loop(0, n_rows, body, 0)


def _zero_acc_slice(zsrc_v, acc_s, s, rows_per_tile):
    for q in range(rows_per_tile // ECHUNK):
        pltpu.sync_copy(
            zsrc_v, acc_s.at[pl.ds(s * rows_per_tile + q * ECHUNK, ECHUNK)])


def _writeback(acc_s, out_hbm, c, s, rows_per_tile):
    npad = acc_s.shape[0]
    pltpu.sync_copy(acc_s.at[pl.ds(s * rows_per_tile, rows_per_tile)],
                    out_hbm.at[pl.ds(c * npad + s * rows_per_tile, rows_per_tile)])


def _deg_body(dst_hbm, out_hbm, ones_v, idx_v, acc_s):
    c = lax.axis_index("c")
    s = lax.axis_index("s")
    w = c * 16 + s
    chunks = idx_v.shape[0]
    rows_per_tile = acc_s.shape[0] // 16

    pltpu.sync_copy(dst_hbm.at[w], idx_v)

    _fill_const(ones_v, ECHUNK, 0.0)
    _zero_acc_slice(ones_v, acc_s, s, rows_per_tile)
    _fill_const(ones_v, ECHUNK, 1.0)
    plsc.subcore_barrier()

    def chunk(j, _):
        pltpu.sync_copy(ones_v, acc_s.at[idx_v.at[j]], add=True)
        return 0
    lax.fori_loop(0, chunks, chunk, 0)
    plsc.subcore_barrier()

    _writeback(acc_s, out_hbm, c, s, rows_per_tile)


def _edge_body(ys_hbm, src_hbm, dst_hbm, out_hbm,
               sidx_v, didx_v, rows_v, acc_s, sem):
    c = lax.axis_index("c")
    s = lax.axis_index("s")
    w = c * 16 + s
    chunks = sidx_v.shape[0]
    rows_per_tile = acc_s.shape[0] // 16

    pltpu.sync_copy(src_hbm.at[w], sidx_v)
    pltpu.sync_copy(dst_hbm.at[w], didx_v)

    _fill_const(rows_v, ECHUNK, 0.0)
    _zero_acc_slice(rows_v, acc_s, s, rows_per_tile)
    plsc.subcore_barrier()

    def chunk(j, _):
        pltpu.async_copy(ys_hbm.at[sidx_v.at[j]], rows_v, sem).wait()
        pltpu.sync_copy(rows_v, acc_s.at[didx_v.at[j]], add=True)
        return 0
    lax.fori_loop(0, chunks, chunk, 0)
    plsc.subcore_barrier()

    _writeback(acc_s, out_hbm, c, s, rows_per_tile)


def _make_sc_calls(npad, chunks):
    mesh = plsc.VectorSubcoreMesh(core_axis_name="c", subcore_axis_name="s")
    deg_call = pl.kernel(
        _deg_body,
        out_type=jax.ShapeDtypeStruct((2 * npad, HID), jnp.float32),
        mesh=mesh,
        scratch_types=[
            pltpu.VMEM((ECHUNK, HID), jnp.float32),
            pltpu.VMEM((chunks, ECHUNK), jnp.int32),
            pltpu.VMEM_SHARED((npad, HID), jnp.float32),
        ],
    )
    edge_call = pl.kernel(
        _edge_body,
        out_type=jax.ShapeDtypeStruct((2 * npad, HID), jnp.float32),
        mesh=mesh,
        scratch_types=[
            pltpu.VMEM((chunks, ECHUNK), jnp.int32),
            pltpu.VMEM((chunks, ECHUNK), jnp.int32),
            pltpu.VMEM((ECHUNK, HID), jnp.float32),
            pltpu.VMEM_SHARED((npad, HID), jnp.float32),
            pltpu.SemaphoreType.DMA,
        ],
    )
    return deg_call, edge_call


def _dinv(degp_ref):
    return lax.rsqrt(degp_ref[0, :, 0] + degp_ref[1, :, 0] + 1.0)


def _prep_body(x_ref, w_ref, degp_ref, o_ref):
    dinv = _dinv(degp_ref)
    y = jnp.dot(x_ref[...], w_ref[...], preferred_element_type=jnp.float32)
    o_ref[...] = y * dinv[:, None]


def _mid_body(acc_ref, ys_ref, degp_ref, b1_ref, w2_ref, o_ref):
    dinv = _dinv(degp_ref)
    h = (acc_ref[0] + acc_ref[1] + ys_ref[...]) * dinv[:, None] + b1_ref[0][None, :]
    h = jnp.maximum(h, 0.0)
    o_ref[...] = jnp.dot(h, w2_ref[...], preferred_element_type=jnp.float32) * dinv[:, None]


def _gi_body(acc_ref, ys_ref, degp_ref, b2_ref, wih_ref, bih_ref, o_ref):
    dinv = _dinv(degp_ref)
    h = (acc_ref[0] + acc_ref[1] + ys_ref[...]) * dinv[:, None] + b2_ref[0][None, :]
    h = jnp.maximum(h, 0.0)
    o_ref[...] = (jnp.dot(h, wih_ref[...], preferred_element_type=jnp.float32)
                  + bih_ref[0][None, :])


TSTEP = 8


def _gru_body(gi_ref, whh_ref, bhh_ref, o_ref, h_ref):
    t = pl.program_id(0)

    @pl.when(t == 0)
    def _():
        h_ref[...] = jnp.zeros_like(h_ref)

    h = h_ref[...]
    whh = whh_ref[...]
    bhh = bhh_ref[0][None, :]
    for i in range(TSTEP):
        g = gi_ref[:, i, :]
        gh = jnp.dot(h, whh, preferred_element_type=jnp.float32) + bhh
        r = jax.nn.sigmoid(g[:, :HID] + gh[:, :HID])
        z = jax.nn.sigmoid(g[:, HID:2 * HID] + gh[:, HID:2 * HID])
        n = jnp.tanh(g[:, 2 * HID:] + r * gh[:, 2 * HID:])
        h = (1.0 - z) * n + z * h
        o_ref[:, i, :] = h
    h_ref[...] = h


def kernel(x, edge_index, W1, b1, W2, b2, W_ih, W_hh, b_ih, b_hh):
    B, T, C = x.shape
    tpad = 512
    npad = B * tpad
    E = edge_index.shape[1]
    chunks = -(-E // (N_TILES * ECHUNK))
    epad = N_TILES * chunks * ECHUNK

    xf = jnp.pad(x, ((0, 0), (0, tpad - T), (0, 0))).reshape(npad, C)
    src = edge_index[0].astype(jnp.int32)
    dst = edge_index[1].astype(jnp.int32)
    srcp = (src // T) * tpad + (src % T)
    dstp = (dst // T) * tpad + (dst % T)
    srcp = jnp.concatenate([srcp, jnp.zeros((epad - E,), jnp.int32)])
    dstp = jnp.concatenate([dstp, jnp.full((epad - E,), T, jnp.int32)])
    src3 = srcp.reshape(N_TILES, chunks, ECHUNK)
    dst3 = dstp.reshape(N_TILES, chunks, ECHUNK)

    deg_call, edge_call = _make_sc_calls(npad, chunks)

    rows_blk = 1024
    grid = npad // rows_blk

    def tc_call(body, out_dim, *ops):
        specs = []
        for op in ops:
            if op.shape[0] == 2 and op.ndim == 3:
                specs.append(pl.BlockSpec((2, rows_blk, op.shape[2]),
                                          lambda i: (0, i, 0)))
            elif op.shape[0] == npad:
                specs.append(pl.BlockSpec((rows_blk, op.shape[1]),
                                          lambda i: (i, 0)))
            else:
                specs.append(pl.BlockSpec(op.shape, lambda i, nd=op.ndim: (0,) * nd))
        return pl.pallas_call(
            body,
            grid=(grid,),
            in_specs=specs,
            out_specs=pl.BlockSpec((rows_blk, out_dim), lambda i: (i, 0)),
            out_shape=jax.ShapeDtypeStruct((npad, out_dim), jnp.float32),
        )(*ops)

    degp = deg_call(dst3).reshape(2, npad, HID)
    ys1 = tc_call(_prep_body, HID, xf, W1, degp)
    acc1 = edge_call(ys1, src3, dst3).reshape(2, npad, HID)
    ys2 = tc_call(_mid_body, HID, acc1, ys1, degp, b1.reshape(1, HID), W2)
    acc2 = edge_call(ys2, src3, dst3).reshape(2, npad, HID)
    gi = tc_call(_gi_body, H3, acc2, ys2, degp, b2.reshape(1, HID),
                 W_ih.T, b_ih.reshape(1, H3))

    gi3 = gi.reshape(B, tpad, H3)

    return pl.pallas_call(
        _gru_body,
        grid=(-(-T // TSTEP),),
        in_specs=[
            pl.BlockSpec((B, TSTEP, H3), lambda t: (0, t, 0)),
            pl.BlockSpec((HID, H3), lambda t: (0, 0)),
            pl.BlockSpec((1, H3), lambda t: (0, 0)),
        ],
        out_specs=pl.BlockSpec((B, TSTEP, HID), lambda t: (0, t, 0)),
        out_shape=jax.ShapeDtypeStruct((B, T, HID), jnp.float32),
        scratch_shapes=[pltpu.VMEM((B, HID), jnp.float32)],
    )(gi3, W_hh.T, b_hh.reshape(1, H3))
